# baseline (speedup 1.0000x reference)
import functools

import jax
import jax.numpy as jnp
from jax import lax
from jax.experimental import pallas as pl
from jax.experimental.pallas import tpu as pltpu
from jax.experimental.pallas import tpu_sc as plsc

B = 16384
D = 32
GROUPS = 128 // D
NROW = 1000000 // GROUPS
NC = 2
NS = 16
NW = NC * NS
BPW = B // NW
CH = 128
NCH = BPW // CH


def _gather_body(user_hbm, item_hbm, su_hbm, ti_hbm, u_out, i_out,
                 idx_u, idx_i, rows, sem):
    c = lax.axis_index("c")
    s = lax.axis_index("s")
    wid = s * NC + c
    base = wid * BPW
    pltpu.sync_copy(user_hbm.at[pl.ds(base, BPW)], idx_u)
    pltpu.sync_copy(item_hbm.at[pl.ds(base, BPW)], idx_i)
    copies = [pltpu.async_copy(
        su_hbm.at[idx_u.at[pl.ds(j * CH, CH)]],
        rows.at[pl.ds(j * CH, CH)], sem) for j in range(NCH)]
    for cp in copies:
        cp.wait()
    pltpu.sync_copy(rows, u_out.at[pl.ds(base, BPW)])
    copies = [pltpu.async_copy(
        ti_hbm.at[idx_i.at[pl.ds(j * CH, CH)]],
        rows.at[pl.ds(j * CH, CH)], sem) for j in range(NCH)]
    for cp in copies:
        cp.wait()
    pltpu.sync_copy(rows, i_out.at[pl.ds(base, BPW)])


@functools.lru_cache(maxsize=1)
def _make_gather():
    mesh = plsc.VectorSubcoreMesh(core_axis_name="c", subcore_axis_name="s")
    return pl.kernel(
        _gather_body,
        out_type=[
            jax.ShapeDtypeStruct((B, 128), jnp.float32),
            jax.ShapeDtypeStruct((B, 128), jnp.float32),
        ],
        mesh=mesh,
        compiler_params=pltpu.CompilerParams(use_tc_tiling_on_sc=False),
        scratch_types=[
            pltpu.VMEM((BPW,), jnp.int32),
            pltpu.VMEM((BPW,), jnp.int32),
            pltpu.VMEM((BPW, 128), jnp.float32),
            pltpu.SemaphoreType.DMA,
        ],
    )


DT_COLS = 4096
DT_GRID = (1000000 + DT_COLS - 1) // DT_COLS


def _detile_body(ut_ref, it_ref, u_out, i_out):
    for ref, out in ((ut_ref, u_out), (it_ref, i_out)):
        x = ref[...]
        y = x.reshape(D, DT_COLS // GROUPS, GROUPS)
        z = jnp.transpose(y, (1, 2, 0))
        out[...] = z.reshape(DT_COLS // GROUPS, 128)


_detile = pl.pallas_call(
    _detile_body,
    grid=(DT_GRID,),
    in_specs=[
        pl.BlockSpec((D, DT_COLS), lambda i: (0, i)),
        pl.BlockSpec((D, DT_COLS), lambda i: (0, i)),
    ],
    out_specs=[
        pl.BlockSpec((DT_COLS // GROUPS, 128), lambda i: (i, 0)),
        pl.BlockSpec((DT_COLS // GROUPS, 128), lambda i: (i, 0)),
    ],
    out_shape=[
        jax.ShapeDtypeStruct((NROW, 128), jnp.float32),
        jax.ShapeDtypeStruct((NROW, 128), jnp.float32),
    ],
)


ROWS_PER_BLK = 2048
GRID = B // ROWS_PER_BLK


def _mlp_body(u_ref, i_ref, su_ref, si_ref, w1, b1, w2, b2, w3, b3, w4, b4,
              out_ref):
    su = su_ref[...]
    si = si_ref[...]
    uraw = u_ref[...]
    iraw = i_ref[...]
    x = jnp.zeros((uraw.shape[0], D), jnp.float32)
    e = jnp.zeros((uraw.shape[0], D), jnp.float32)
    for k in range(GROUPS):
        x = x + jnp.where(su == k, uraw[:, k * D:(k + 1) * D], 0.0)
        e = e + jnp.where(si == k, iraw[:, k * D:(k + 1) * D], 0.0)
    h = jnp.maximum(
        jnp.dot(x, w1[...], preferred_element_type=jnp.float32) + b1[...], 0.0)
    h = jnp.maximum(
        jnp.dot(h, w2[...], preferred_element_type=jnp.float32) + b2[...], 0.0)
    h = jnp.maximum(
        jnp.dot(h, w3[...], preferred_element_type=jnp.float32) + b3[...], 0.0)
    y = jnp.dot(h, w4[...], preferred_element_type=jnp.float32) + b4[...]
    s = jnp.sum(y * e, axis=1, keepdims=True) * (1.0 / 3.0)
    out_ref[...] = s


def _full(shape):
    return pl.BlockSpec(shape, lambda i: (0, 0))


_mlp = pl.pallas_call(
    _mlp_body,
    grid=(GRID,),
    in_specs=[
        pl.BlockSpec((ROWS_PER_BLK, 128), lambda i: (i, 0)),
        pl.BlockSpec((ROWS_PER_BLK, 128), lambda i: (i, 0)),
        pl.BlockSpec((ROWS_PER_BLK, 1), lambda i: (i, 0)),
        pl.BlockSpec((ROWS_PER_BLK, 1), lambda i: (i, 0)),
        _full((D, 48)), _full((1, 48)),
        _full((48, 48)), _full((1, 48)),
        _full((48, 48)), _full((1, 48)),
        _full((48, D)), _full((1, D)),
    ],
    out_specs=pl.BlockSpec((ROWS_PER_BLK, 1), lambda i: (i, 0)),
    out_shape=jax.ShapeDtypeStruct((B, 1), jnp.float32),
)


def _block_diag3(a, b, c):
    n = a.shape[0]
    z = jnp.zeros((n, n), jnp.float32)
    return jnp.concatenate([
        jnp.concatenate([a, z, z], axis=1),
        jnp.concatenate([z, b, z], axis=1),
        jnp.concatenate([z, z, c], axis=1),
    ], axis=0)


def kernel(user, item, su_table, ti_table, mlp1, mlp2, mlp3):
    user = user.astype(jnp.int32)
    item = item.astype(jnp.int32)
    sur, tir = _detile(su_table.T, ti_table.T)
    u_raw, i_raw = _make_gather()(
        user // GROUPS, item // GROUPS, sur, tir)
    su = (user % GROUPS).reshape(B, 1)
    si = (item % GROUPS).reshape(B, 1)

    (w1a, b1a), (w2a, b2a), (w3a, b3a), (w4a, b4a) = mlp1
    (w1b, b1b), (w2b, b2b), (w3b, b3b), (w4b, b4b) = mlp2
    (w1c, b1c), (w2c, b2c), (w3c, b3c), (w4c, b4c) = mlp3

    W1 = jnp.concatenate([w1a, w1b, w1c], axis=1)
    B1 = jnp.concatenate([b1a, b1b, b1c]).reshape(1, 48)
    W2 = _block_diag3(w2a, w2b, w2c)
    B2 = jnp.concatenate([b2a, b2b, b2c]).reshape(1, 48)
    W3 = _block_diag3(w3a, w3b, w3c)
    B3 = jnp.concatenate([b3a, b3b, b3c]).reshape(1, 48)
    W4 = jnp.concatenate([w4a, w4b, w4c], axis=0)
    B4 = (b4a + b4b + b4c).reshape(1, D)

    score = _mlp(u_raw, i_raw, su, si, W1, B1, W2, B2, W3, B3, W4, B4)
    return score.reshape(B)


# speedup vs baseline: 5.3796x; 5.3796x over previous
import functools

import jax
import jax.numpy as jnp
from jax import lax
from jax.experimental import pallas as pl
from jax.experimental.pallas import tpu as pltpu
from jax.experimental.pallas import tpu_sc as plsc

B = 16384
D = 32
N = 1000000
NC = 2
NS = 16
NW = NC * NS
BPW = B // NW
CH = 128
NCH = BPW // CH

DT_COLS = 4096
DT_GRID = (N + DT_COLS - 1) // DT_COLS


def _detile_body(ut_ref, it_ref, u_out, i_out):
    u_out[...] = ut_ref[...].T
    i_out[...] = it_ref[...].T


_detile = pl.pallas_call(
    _detile_body,
    grid=(DT_GRID,),
    in_specs=[
        pl.BlockSpec((D, DT_COLS), lambda i: (0, i)),
        pl.BlockSpec((D, DT_COLS), lambda i: (0, i)),
    ],
    out_specs=[
        pl.BlockSpec((DT_COLS, D), lambda i: (i, 0)),
        pl.BlockSpec((DT_COLS, D), lambda i: (i, 0)),
    ],
    out_shape=[
        jax.ShapeDtypeStruct((N, D), jnp.float32),
        jax.ShapeDtypeStruct((N, D), jnp.float32),
    ],
)


def _gather_body(user_hbm, item_hbm, su_hbm, ti_hbm, u_out, i_out,
                 idx_u, idx_i, rows_u, rows_i, sem_u, sem_i):
    c = lax.axis_index("c")
    s = lax.axis_index("s")
    wid = s * NC + c
    base = wid * BPW
    pltpu.sync_copy(user_hbm.at[pl.ds(base, BPW)], idx_u)
    pltpu.sync_copy(item_hbm.at[pl.ds(base, BPW)], idx_i)
    copies = []
    for j in range(NCH):
        copies.append(pltpu.async_copy(
            su_hbm.at[idx_u.at[pl.ds(j * CH, CH)]],
            rows_u.at[pl.ds(j * CH, CH)], sem_u))
    for j in range(NCH):
        copies.append(pltpu.async_copy(
            ti_hbm.at[idx_i.at[pl.ds(j * CH, CH)]],
            rows_i.at[pl.ds(j * CH, CH)], sem_i))
    for cp in copies:
        cp.wait()
    pltpu.sync_copy(rows_u, u_out.at[pl.ds(base, BPW)])
    pltpu.sync_copy(rows_i, i_out.at[pl.ds(base, BPW)])


@functools.lru_cache(maxsize=1)
def _make_gather():
    mesh = plsc.VectorSubcoreMesh(core_axis_name="c", subcore_axis_name="s")
    return pl.kernel(
        _gather_body,
        out_type=[
            jax.ShapeDtypeStruct((B, D), jnp.float32),
            jax.ShapeDtypeStruct((B, D), jnp.float32),
        ],
        mesh=mesh,
        compiler_params=pltpu.CompilerParams(use_tc_tiling_on_sc=False),
        scratch_types=[
            pltpu.VMEM((BPW,), jnp.int32),
            pltpu.VMEM((BPW,), jnp.int32),
            pltpu.VMEM((BPW, D), jnp.float32),
            pltpu.VMEM((BPW, D), jnp.float32),
            pltpu.SemaphoreType.DMA,
            pltpu.SemaphoreType.DMA,
        ],
    )


ROWS_PER_BLK = 2048
GRID = B // ROWS_PER_BLK


def _mlp_body(u_ref, i_ref, w1, b1, w2, b2, w3, b3, w4, b4, out_ref):
    x = u_ref[...]
    h = jnp.maximum(
        jnp.dot(x, w1[...], preferred_element_type=jnp.float32) + b1[...], 0.0)
    h = jnp.maximum(
        jnp.dot(h, w2[...], preferred_element_type=jnp.float32) + b2[...], 0.0)
    h = jnp.maximum(
        jnp.dot(h, w3[...], preferred_element_type=jnp.float32) + b3[...], 0.0)
    y = jnp.dot(h, w4[...], preferred_element_type=jnp.float32) + b4[...]
    s = jnp.sum(y * i_ref[...], axis=1, keepdims=True) * (1.0 / 3.0)
    out_ref[...] = s


def _full(shape):
    return pl.BlockSpec(shape, lambda i: (0, 0))


_mlp = pl.pallas_call(
    _mlp_body,
    grid=(GRID,),
    in_specs=[
        pl.BlockSpec((ROWS_PER_BLK, D), lambda i: (i, 0)),
        pl.BlockSpec((ROWS_PER_BLK, D), lambda i: (i, 0)),
        _full((D, 48)), _full((1, 48)),
        _full((48, 48)), _full((1, 48)),
        _full((48, 48)), _full((1, 48)),
        _full((48, D)), _full((1, D)),
    ],
    out_specs=pl.BlockSpec((ROWS_PER_BLK, 1), lambda i: (i, 0)),
    out_shape=jax.ShapeDtypeStruct((B, 1), jnp.float32),
)


def _block_diag3(a, b, c):
    n = a.shape[0]
    z = jnp.zeros((n, n), jnp.float32)
    return jnp.concatenate([
        jnp.concatenate([a, z, z], axis=1),
        jnp.concatenate([z, b, z], axis=1),
        jnp.concatenate([z, z, c], axis=1),
    ], axis=0)


def kernel(user, item, su_table, ti_table, mlp1, mlp2, mlp3):
    u_emb, i_emb = _make_gather()(
        user.astype(jnp.int32), item.astype(jnp.int32), su_table, ti_table)

    (w1a, b1a), (w2a, b2a), (w3a, b3a), (w4a, b4a) = mlp1
    (w1b, b1b), (w2b, b2b), (w3b, b3b), (w4b, b4b) = mlp2
    (w1c, b1c), (w2c, b2c), (w3c, b3c), (w4c, b4c) = mlp3

    W1 = jnp.concatenate([w1a, w1b, w1c], axis=1)
    B1 = jnp.concatenate([b1a, b1b, b1c]).reshape(1, 48)
    W2 = _block_diag3(w2a, w2b, w2c)
    B2 = jnp.concatenate([b2a, b2b, b2c]).reshape(1, 48)
    W3 = _block_diag3(w3a, w3b, w3c)
    B3 = jnp.concatenate([b3a, b3b, b3c]).reshape(1, 48)
    W4 = jnp.concatenate([w4a, w4b, w4c], axis=0)
    B4 = (b4a + b4b + b4c).reshape(1, D)

    score = _mlp(u_emb, i_emb, W1, B1, W2, B2, W3, B3, W4, B4)
    return score.reshape(B)


# speedup vs baseline: 5.3826x; 1.0006x over previous
import functools

import jax
import jax.numpy as jnp
from jax import lax
from jax.experimental import pallas as pl
from jax.experimental.pallas import tpu as pltpu
from jax.experimental.pallas import tpu_sc as plsc

B = 16384
D = 32
N = 1000000
NC = 2
NS = 16
NW = NC * NS
BPW = B // NW
CH = 512
NCH = BPW // CH

DT_COLS = 4096
DT_GRID = (N + DT_COLS - 1) // DT_COLS


def _detile_body(ut_ref, it_ref, u_out, i_out):
    u_out[...] = ut_ref[...].T
    i_out[...] = it_ref[...].T


_detile = pl.pallas_call(
    _detile_body,
    grid=(DT_GRID,),
    in_specs=[
        pl.BlockSpec((D, DT_COLS), lambda i: (0, i)),
        pl.BlockSpec((D, DT_COLS), lambda i: (0, i)),
    ],
    out_specs=[
        pl.BlockSpec((DT_COLS, D), lambda i: (i, 0)),
        pl.BlockSpec((DT_COLS, D), lambda i: (i, 0)),
    ],
    out_shape=[
        jax.ShapeDtypeStruct((N, D), jnp.float32),
        jax.ShapeDtypeStruct((N, D), jnp.float32),
    ],
)


def _gather_body(user_hbm, item_hbm, su_hbm, ti_hbm, u_out, i_out,
                 idx_u, idx_i, rows_u, rows_i, sem_u, sem_i):
    c = lax.axis_index("c")
    s = lax.axis_index("s")
    wid = s * NC + c
    base = wid * BPW
    pltpu.sync_copy(user_hbm.at[pl.ds(base, BPW)], idx_u)
    pltpu.sync_copy(item_hbm.at[pl.ds(base, BPW)], idx_i)
    copies = []
    for j in range(NCH):
        copies.append(pltpu.async_copy(
            su_hbm.at[idx_u.at[pl.ds(j * CH, CH)]],
            rows_u.at[pl.ds(j * CH, CH)], sem_u))
    for j in range(NCH):
        copies.append(pltpu.async_copy(
            ti_hbm.at[idx_i.at[pl.ds(j * CH, CH)]],
            rows_i.at[pl.ds(j * CH, CH)], sem_i))
    for cp in copies:
        cp.wait()
    pltpu.sync_copy(rows_u, u_out.at[pl.ds(base, BPW)])
    pltpu.sync_copy(rows_i, i_out.at[pl.ds(base, BPW)])


@functools.lru_cache(maxsize=1)
def _make_gather():
    mesh = plsc.VectorSubcoreMesh(core_axis_name="c", subcore_axis_name="s")
    return pl.kernel(
        _gather_body,
        out_type=[
            jax.ShapeDtypeStruct((B, D), jnp.float32),
            jax.ShapeDtypeStruct((B, D), jnp.float32),
        ],
        mesh=mesh,
        compiler_params=pltpu.CompilerParams(use_tc_tiling_on_sc=False),
        scratch_types=[
            pltpu.VMEM((BPW,), jnp.int32),
            pltpu.VMEM((BPW,), jnp.int32),
            pltpu.VMEM((BPW, D), jnp.float32),
            pltpu.VMEM((BPW, D), jnp.float32),
            pltpu.SemaphoreType.DMA,
            pltpu.SemaphoreType.DMA,
        ],
    )


ROWS_PER_BLK = 2048
GRID = B // ROWS_PER_BLK


def _mlp_body(u_ref, i_ref, w1, b1, w2, b2, w3, b3, w4, b4, out_ref):
    x = u_ref[...]
    h = jnp.maximum(
        jnp.dot(x, w1[...], preferred_element_type=jnp.float32) + b1[...], 0.0)
    h = jnp.maximum(
        jnp.dot(h, w2[...], preferred_element_type=jnp.float32) + b2[...], 0.0)
    h = jnp.maximum(
        jnp.dot(h, w3[...], preferred_element_type=jnp.float32) + b3[...], 0.0)
    y = jnp.dot(h, w4[...], preferred_element_type=jnp.float32) + b4[...]
    s = jnp.sum(y * i_ref[...], axis=1, keepdims=True) * (1.0 / 3.0)
    out_ref[...] = s


def _full(shape):
    return pl.BlockSpec(shape, lambda i: (0, 0))


_mlp = pl.pallas_call(
    _mlp_body,
    grid=(GRID,),
    in_specs=[
        pl.BlockSpec((ROWS_PER_BLK, D), lambda i: (i, 0)),
        pl.BlockSpec((ROWS_PER_BLK, D), lambda i: (i, 0)),
        _full((D, 48)), _full((1, 48)),
        _full((48, 48)), _full((1, 48)),
        _full((48, 48)), _full((1, 48)),
        _full((48, D)), _full((1, D)),
    ],
    out_specs=pl.BlockSpec((ROWS_PER_BLK, 1), lambda i: (i, 0)),
    out_shape=jax.ShapeDtypeStruct((B, 1), jnp.float32),
)


def _block_diag3(a, b, c):
    n = a.shape[0]
    z = jnp.zeros((n, n), jnp.float32)
    return jnp.concatenate([
        jnp.concatenate([a, z, z], axis=1),
        jnp.concatenate([z, b, z], axis=1),
        jnp.concatenate([z, z, c], axis=1),
    ], axis=0)


def kernel(user, item, su_table, ti_table, mlp1, mlp2, mlp3):
    u_emb, i_emb = _make_gather()(
        user.astype(jnp.int32), item.astype(jnp.int32), su_table, ti_table)

    (w1a, b1a), (w2a, b2a), (w3a, b3a), (w4a, b4a) = mlp1
    (w1b, b1b), (w2b, b2b), (w3b, b3b), (w4b, b4b) = mlp2
    (w1c, b1c), (w2c, b2c), (w3c, b3c), (w4c, b4c) = mlp3

    W1 = jnp.concatenate([w1a, w1b, w1c], axis=1)
    B1 = jnp.concatenate([b1a, b1b, b1c]).reshape(1, 48)
    W2 = _block_diag3(w2a, w2b, w2c)
    B2 = jnp.concatenate([b2a, b2b, b2c]).reshape(1, 48)
    W3 = _block_diag3(w3a, w3b, w3c)
    B3 = jnp.concatenate([b3a, b3b, b3c]).reshape(1, 48)
    W4 = jnp.concatenate([w4a, w4b, w4c], axis=0)
    B4 = (b4a + b4b + b4c).reshape(1, D)

    score = _mlp(u_emb, i_emb, W1, B1, W2, B2, W3, B3, W4, B4)
    return score.reshape(B)
